# Initial kernel scaffold; baseline (speedup 1.0000x reference)
import jax
import jax.numpy as jnp
from jax.experimental import pallas as pl


def kernel(data, length, W):
    raise NotImplementedError("write your pallas kernel here")


# speedup vs baseline: 2.3266x; 2.3266x over previous
import functools

import jax
import jax.numpy as jnp
from jax import lax
from jax.experimental import pallas as pl
from jax.experimental.pallas import tpu as pltpu
from jax.experimental.pallas import tpu_sc as plsc

_LANE = 16
_NBUF = 2


def _cbow_sc(B, L, E):
    info = plsc.get_sparse_core_info()
    NC, NS = info.num_cores, info.num_subcores
    NW = NC * NS
    assert B % NW == 0
    BPW = B // NW
    EC = E // _LANE
    inv = 1.0 / L

    @functools.partial(
        pl.kernel,
        out_type=jax.ShapeDtypeStruct((B, E), jnp.float32),
        mesh=plsc.VectorSubcoreMesh(core_axis_name="c", subcore_axis_name="s"),
        compiler_params=pltpu.CompilerParams(use_tc_tiling_on_sc=False),
        scratch_types=[
            pltpu.VMEM((BPW, L), jnp.int32),
            pltpu.VMEM((_NBUF, L, E), jnp.float32),
            pltpu.VMEM((BPW, E), jnp.float32),
            pltpu.SemaphoreType.DMA,
            pltpu.SemaphoreType.DMA,
        ],
    )
    def cbow_kernel(data_hbm, w_hbm, out_hbm, idx_v, rows_v, out_v, sem0, sem1):
        sems = (sem0, sem1)
        wid = lax.axis_index("s") * NC + lax.axis_index("c")
        base = wid * BPW
        pltpu.sync_copy(data_hbm.at[pl.ds(base, BPW)], idx_v)
        for b in range(_NBUF):
            pltpu.async_copy(w_hbm.at[idx_v.at[b]], rows_v.at[b], sems[b])

        def outer(g, carry):
            for b in range(_NBUF):
                k = g * _NBUF + b
                pltpu.make_async_copy(
                    w_hbm.at[idx_v.at[k]], rows_v.at[b], sems[b]
                ).wait()

                def red(r, acc, b=b):
                    return tuple(
                        acc[c] + rows_v[b, r, pl.ds(c * _LANE, _LANE)]
                        for c in range(EC)
                    )

                acc = lax.fori_loop(
                    0, L, red,
                    tuple(jnp.zeros((_LANE,), jnp.float32) for _ in range(EC)),
                )
                for c in range(EC):
                    out_v[k, pl.ds(c * _LANE, _LANE)] = acc[c] * inv
                nk = k + _NBUF

                @pl.when(nk < BPW)
                def _(b=b, nk=nk):
                    pltpu.async_copy(w_hbm.at[idx_v.at[nk]], rows_v.at[b], sems[b])

            return carry

        lax.fori_loop(0, BPW // _NBUF, outer, 0)
        pltpu.sync_copy(out_v, out_hbm.at[pl.ds(base, BPW)])

    return cbow_kernel


def kernel(data, length, W):
    B, L = data.shape
    V, E = W.shape
    out = _cbow_sc(B, L, E)(data.astype(jnp.int32), W)
    return out


# speedup vs baseline: 2.5598x; 1.1002x over previous
import functools

import jax
import jax.numpy as jnp
from jax import lax
from jax.experimental import pallas as pl
from jax.experimental.pallas import tpu as pltpu
from jax.experimental.pallas import tpu_sc as plsc

_LANE = 16
_NBUF = 2
_G = 2


def _cbow_sc(B, L, E):
    info = plsc.get_sparse_core_info()
    NC, NS = info.num_cores, info.num_subcores
    NW = NC * NS
    assert B % (NW * _G) == 0
    BPW = B // NW
    NCHUNK = BPW // _G
    EC = E // _LANE
    inv = 1.0 / L

    @functools.partial(
        pl.kernel,
        out_type=jax.ShapeDtypeStruct((B, E), jnp.float32),
        mesh=plsc.VectorSubcoreMesh(core_axis_name="c", subcore_axis_name="s"),
        compiler_params=pltpu.CompilerParams(use_tc_tiling_on_sc=False),
        scratch_types=[
            pltpu.VMEM((NCHUNK, _G * L), jnp.int32),
            pltpu.VMEM((_NBUF, _G * L, E), jnp.float32),
            pltpu.VMEM((BPW, E), jnp.float32),
        ]
        + [pltpu.SemaphoreType.DMA] * _NBUF,
    )
    def cbow_kernel(data_hbm, w_hbm, out_hbm, idx_v, rows_v, out_v, *sems):
        wid = lax.axis_index("s") * NC + lax.axis_index("c")
        base = wid * NCHUNK
        pltpu.sync_copy(data_hbm.at[pl.ds(base, NCHUNK)], idx_v)
        for b in range(_NBUF):
            pltpu.async_copy(w_hbm.at[idx_v.at[b]], rows_v.at[b], sems[b])

        def outer(g, carry):
            for b in range(_NBUF):
                k = g * _NBUF + b
                pltpu.make_async_copy(
                    w_hbm.at[idx_v.at[k]], rows_v.at[b], sems[b]
                ).wait()

                for e in range(_G):
                    def red(r, acc, b=b, e=e):
                        return tuple(
                            acc[c] + rows_v[b, e * L + r, pl.ds(c * _LANE, _LANE)]
                            for c in range(EC)
                        )

                    acc = lax.fori_loop(
                        0, L, red,
                        tuple(jnp.zeros((_LANE,), jnp.float32) for _ in range(EC)),
                    )
                    for c in range(EC):
                        out_v[k * _G + e, pl.ds(c * _LANE, _LANE)] = acc[c] * inv

                nk = k + _NBUF

                @pl.when(nk < NCHUNK)
                def _(b=b, nk=nk):
                    pltpu.async_copy(w_hbm.at[idx_v.at[nk]], rows_v.at[b], sems[b])

            return carry

        lax.fori_loop(0, NCHUNK // _NBUF, outer, 0)
        pltpu.sync_copy(out_v, out_hbm.at[pl.ds(wid * BPW, BPW)])

    return cbow_kernel


def kernel(data, length, W):
    B, L = data.shape
    V, E = W.shape
    data_r = data.astype(jnp.int32).reshape(B // _G, _G * L)
    out = _cbow_sc(B, L, E)(data_r, W)
    return out


# speedup vs baseline: 3.3102x; 1.2932x over previous
import functools

import jax
import jax.numpy as jnp
from jax import lax
from jax.experimental import pallas as pl
from jax.experimental.pallas import tpu as pltpu
from jax.experimental.pallas import tpu_sc as plsc

_LANE = 16
_NBUF = 2
_G = 2


def _cbow_sc(B, L, E, V):
    info = plsc.get_sparse_core_info()
    NC, NS = info.num_cores, info.num_subcores
    NW = NC * NS
    assert B % (NW * _G) == 0
    BPW = B // NW
    NCHUNK = BPW // _G
    EC = E // _LANE
    inv = 1.0 / L

    @functools.partial(
        pl.kernel,
        out_type=jax.ShapeDtypeStruct((B, E), jnp.float32),
        mesh=plsc.VectorSubcoreMesh(core_axis_name="c", subcore_axis_name="s"),
        compiler_params=pltpu.CompilerParams(use_tc_tiling_on_sc=False),
        scratch_types=[
            pltpu.VMEM((NCHUNK, _G * L), jnp.int32),
            pltpu.VMEM((_NBUF, _G * L, E), jnp.float32),
            pltpu.VMEM((BPW, E), jnp.float32),
        ]
        + [pltpu.SemaphoreType.DMA] * _NBUF,
    )
    def cbow_kernel(data_hbm, w_hbm, out_hbm, idx_v, rows_v, out_v, *sems):
        wid = lax.axis_index("s") * NC + lax.axis_index("c")
        base = wid * NCHUNK
        pltpu.sync_copy(data_hbm.at[pl.ds(base, NCHUNK)], idx_v)
        for b in range(_NBUF):
            pltpu.async_copy(w_hbm.at[idx_v.at[b]], rows_v.at[b], sems[b])

        def outer(g, carry):
            for b in range(_NBUF):
                k = g * _NBUF + b
                pltpu.make_async_copy(
                    w_hbm.at[idx_v.at[k]], rows_v.at[b], sems[b]
                ).wait()

                for e in range(_G):
                    def red(r, acc, b=b, e=e):
                        return tuple(
                            acc[c] + rows_v[b, e * L + r, pl.ds(c * _LANE, _LANE)]
                            for c in range(EC)
                        )

                    acc = lax.fori_loop(
                        0, L, red,
                        tuple(jnp.zeros((_LANE,), jnp.float32) for _ in range(EC)),
                    )
                    for c in range(EC):
                        out_v[k * _G + e, pl.ds(c * _LANE, _LANE)] = acc[c] * inv

                nk = k + _NBUF

                @pl.when(nk < NCHUNK)
                def _(b=b, nk=nk):
                    pltpu.async_copy(w_hbm.at[idx_v.at[nk]], rows_v.at[b], sems[b])

            return carry

        lax.fori_loop(0, NCHUNK // _NBUF, outer, 0)
        pltpu.sync_copy(out_v, out_hbm.at[pl.ds(wid * BPW, BPW)])

    return cbow_kernel


def _relayout_tc(V, E, C=8192):
    grid = (V + C - 1) // C

    def body(wt_ref, out_ref):
        y = wt_ref[...].T
        z = y.reshape(C // 2, 2, E)
        out_ref[...] = jnp.concatenate([z[:, 0, :], z[:, 1, :]], axis=1)

    return pl.pallas_call(
        body,
        grid=(grid,),
        in_specs=[pl.BlockSpec((E, C), lambda i: (0, i))],
        out_specs=pl.BlockSpec((C // 2, 2 * E), lambda i: (i, 0)),
        out_shape=jax.ShapeDtypeStruct((V // 2, 2 * E), jnp.float32),
    )


def kernel(data, length, W):
    B, L = data.shape
    V, E = W.shape
    data_r = data.astype(jnp.int32).reshape(B // _G, _G * L)
    w128 = _relayout_tc(V, E)(W.T)
    w_lin = w128.reshape(V, E)
    out = _cbow_sc(B, L, E, V)(data_r, w_lin)
    return out


# speedup vs baseline: 3.7485x; 1.1324x over previous
import functools

import jax
import jax.numpy as jnp
from jax import lax
from jax.experimental import pallas as pl
from jax.experimental.pallas import tpu as pltpu
from jax.experimental.pallas import tpu_sc as plsc

_LANE = 16
_NBUF = 2
_G = 2


def _cbow_sc(B, L, E, V):
    info = plsc.get_sparse_core_info()
    NC, NS = info.num_cores, info.num_subcores
    NW = NC * NS
    assert B % (NW * _G) == 0
    BPW = B // NW
    NCHUNK = BPW // _G
    EC = E // _LANE
    inv = 1.0 / L

    @functools.partial(
        pl.kernel,
        out_type=jax.ShapeDtypeStruct((B, E), jnp.float32),
        mesh=plsc.VectorSubcoreMesh(core_axis_name="c", subcore_axis_name="s"),
        compiler_params=pltpu.CompilerParams(use_tc_tiling_on_sc=False),
        scratch_types=[
            pltpu.VMEM((NCHUNK, _G * L), jnp.int32),
            pltpu.VMEM((_NBUF, _G * L, E), jnp.float32),
            pltpu.VMEM((BPW, E), jnp.float32),
        ]
        + [pltpu.SemaphoreType.DMA] * _NBUF,
    )
    def cbow_kernel(data_hbm, w_hbm, out_hbm, idx_v, rows_v, out_v, *sems):
        wid = lax.axis_index("s") * NC + lax.axis_index("c")
        base = wid * NCHUNK
        pltpu.sync_copy(data_hbm.at[pl.ds(base, NCHUNK)], idx_v)
        for b in range(_NBUF):
            pltpu.async_copy(w_hbm.at[idx_v.at[b]], rows_v.at[b], sems[b])

        def outer(g, carry):
            for b in range(_NBUF):
                k = g * _NBUF + b
                pltpu.make_async_copy(
                    w_hbm.at[idx_v.at[k]], rows_v.at[b], sems[b]
                ).wait()

                for e in range(_G):
                    def red(r, acc, b=b, e=e):
                        return tuple(
                            acc[c] + rows_v[b, e * L + r, pl.ds(c * _LANE, _LANE)]
                            for c in range(EC)
                        )

                    acc = lax.fori_loop(
                        0, L, red,
                        tuple(jnp.zeros((_LANE,), jnp.float32) for _ in range(EC)),
                    )
                    for c in range(EC):
                        out_v[k * _G + e, pl.ds(c * _LANE, _LANE)] = acc[c] * inv

                nk = k + _NBUF

                @pl.when(nk < NCHUNK)
                def _(b=b, nk=nk):
                    pltpu.async_copy(w_hbm.at[idx_v.at[nk]], rows_v.at[b], sems[b])

            return carry

        lax.fori_loop(0, NCHUNK // _NBUF, outer, 0)
        pltpu.sync_copy(out_v, out_hbm.at[pl.ds(wid * BPW, BPW)])

    return cbow_kernel


_K = 16


def _relayout_tc(V, E):
    grid = (V + 256 * _K - 1) // (256 * _K)

    def body(wt_ref, out_ref):
        for k in range(_K):
            x = wt_ref[:, pl.ds(256 * k, 256)]
            out_ref[pl.ds(128 * k, 128), :] = jnp.concatenate(
                [x[:, 0:128].T, x[:, 128:256].T], axis=1
            )

    rows_out = grid * 128 * _K
    return pl.pallas_call(
        body,
        grid=(grid,),
        in_specs=[pl.BlockSpec((E, 256 * _K), lambda i: (0, i))],
        out_specs=pl.BlockSpec((128 * _K, 2 * E), lambda i: (i, 0)),
        out_shape=jax.ShapeDtypeStruct((rows_out, 2 * E), jnp.float32),
    )


def kernel(data, length, W):
    B, L = data.shape
    V, E = W.shape
    d = data.astype(jnp.int32)
    d = (d & ~jnp.int32(255)) | ((d & 127) << 1) | ((d >> 7) & 1)
    data_r = d.reshape(B // _G, _G * L)
    w128 = _relayout_tc(V, E)(W.T)
    w_lin = w128.reshape(w128.shape[0] * 2, E)
    out = _cbow_sc(B, L, E, w_lin.shape[0])(data_r, w_lin)
    return out


# speedup vs baseline: 4.2836x; 1.1427x over previous
import functools

import jax
import jax.numpy as jnp
from jax import lax
from jax.experimental import pallas as pl
from jax.experimental.pallas import tpu as pltpu
from jax.experimental.pallas import tpu_sc as plsc

_LANE = 16
_NBUF = 2
_G = 2


def _cbow_sc(B, L, E, V):
    info = plsc.get_sparse_core_info()
    NC, NS = info.num_cores, info.num_subcores
    NW = NC * NS
    assert B % (NW * _G) == 0
    BPW = B // NW
    NCHUNK = BPW // _G
    EC = E // _LANE
    inv = 1.0 / L

    @functools.partial(
        pl.kernel,
        out_type=jax.ShapeDtypeStruct((B, E), jnp.float32),
        mesh=plsc.VectorSubcoreMesh(core_axis_name="c", subcore_axis_name="s"),
        compiler_params=pltpu.CompilerParams(use_tc_tiling_on_sc=False),
        scratch_types=[
            pltpu.VMEM((NCHUNK, _G * L), jnp.int32),
            pltpu.VMEM((_NBUF, _G * L, E), jnp.float32),
            pltpu.VMEM((BPW, E), jnp.float32),
        ]
        + [pltpu.SemaphoreType.DMA] * _NBUF,
    )
    def cbow_kernel(data_hbm, w_hbm, out_hbm, idx_v, rows_v, out_v, *sems):
        wid = lax.axis_index("s") * NC + lax.axis_index("c")
        base = wid * NCHUNK
        pltpu.sync_copy(data_hbm.at[pl.ds(base, NCHUNK)], idx_v)
        for b in range(_NBUF):
            pltpu.async_copy(w_hbm.at[idx_v.at[b]], rows_v.at[b], sems[b])

        def outer(g, carry):
            for b in range(_NBUF):
                k = g * _NBUF + b
                pltpu.make_async_copy(
                    w_hbm.at[idx_v.at[k]], rows_v.at[b], sems[b]
                ).wait()

                for e in range(_G):
                    def red(r, acc, b=b, e=e):
                        return tuple(
                            acc[c] + rows_v[b, e * L + r, pl.ds(c * _LANE, _LANE)]
                            for c in range(EC)
                        )

                    acc = lax.fori_loop(
                        0, L, red,
                        tuple(jnp.zeros((_LANE,), jnp.float32) for _ in range(EC)),
                    )
                    for c in range(EC):
                        out_v[k * _G + e, pl.ds(c * _LANE, _LANE)] = acc[c] * inv

                nk = k + _NBUF

                @pl.when(nk < NCHUNK)
                def _(b=b, nk=nk):
                    pltpu.async_copy(w_hbm.at[idx_v.at[nk]], rows_v.at[b], sems[b])

            return carry

        lax.fori_loop(0, NCHUNK // _NBUF, outer, 0)
        pltpu.sync_copy(out_v, out_hbm.at[pl.ds(wid * BPW, BPW)])

    return cbow_kernel


_K = 32


def _relayout_tc(V, E):
    grid = (V + 256 * _K - 1) // (256 * _K)

    def body(wt_ref, out_ref):
        for k in range(_K):
            x = wt_ref[:, pl.ds(256 * k, 256)]
            out_ref[pl.ds(128 * k, 128), :] = jnp.concatenate(
                [x[:, 0:128].T, x[:, 128:256].T], axis=1
            )

    rows_out = grid * 128 * _K
    return pl.pallas_call(
        body,
        grid=(grid,),
        in_specs=[pl.BlockSpec((E, 256 * _K), lambda i: (0, i))],
        out_specs=pl.BlockSpec((128 * _K, 2 * E), lambda i: (i, 0)),
        out_shape=jax.ShapeDtypeStruct((rows_out, 2 * E), jnp.float32),
    )


def kernel(data, length, W):
    B, L = data.shape
    V, E = W.shape
    d = data.astype(jnp.int32)
    d = (d & ~jnp.int32(255)) | ((d & 127) << 1) | ((d >> 7) & 1)
    data_r = d.reshape(B // _G, _G * L)
    w128 = _relayout_tc(V, E)(W.T)
    w_lin = w128.reshape(w128.shape[0] * 2, E)
    out = _cbow_sc(B, L, E, w_lin.shape[0])(data_r, w_lin)
    return out


# speedup vs baseline: 4.5913x; 1.0719x over previous
import functools

import jax
import jax.numpy as jnp
from jax import lax
from jax.experimental import pallas as pl
from jax.experimental.pallas import tpu as pltpu
from jax.experimental.pallas import tpu_sc as plsc

_LANE = 16
_NBUF = 2
_G = 2


def _cbow_sc(B, L, E, V):
    info = plsc.get_sparse_core_info()
    NC, NS = info.num_cores, info.num_subcores
    NW = NC * NS
    assert B % (NW * _G) == 0
    BPW = B // NW
    NCHUNK = BPW // _G
    EC = E // _LANE
    inv = 1.0 / L

    @functools.partial(
        pl.kernel,
        out_type=jax.ShapeDtypeStruct((B, E), jnp.float32),
        mesh=plsc.VectorSubcoreMesh(core_axis_name="c", subcore_axis_name="s"),
        compiler_params=pltpu.CompilerParams(use_tc_tiling_on_sc=False),
        scratch_types=[
            pltpu.VMEM((NCHUNK, _G * L), jnp.int32),
            pltpu.VMEM((_NBUF, _G * L, E), jnp.float32),
            pltpu.VMEM((BPW, E), jnp.float32),
        ]
        + [pltpu.SemaphoreType.DMA] * _NBUF,
    )
    def cbow_kernel(data_hbm, w_hbm, out_hbm, idx_v, rows_v, out_v, *sems):
        wid = lax.axis_index("s") * NC + lax.axis_index("c")
        base = wid * NCHUNK
        pltpu.sync_copy(data_hbm.at[pl.ds(base, NCHUNK)], idx_v)
        for b in range(_NBUF):
            pltpu.async_copy(w_hbm.at[idx_v.at[b]], rows_v.at[b], sems[b])

        def outer(g, carry):
            for b in range(_NBUF):
                k = g * _NBUF + b
                pltpu.make_async_copy(
                    w_hbm.at[idx_v.at[k]], rows_v.at[b], sems[b]
                ).wait()

                for e in range(_G):
                    def red(r, acc, b=b, e=e):
                        return tuple(
                            acc[c] + rows_v[b, e * L + r, pl.ds(c * _LANE, _LANE)]
                            for c in range(EC)
                        )

                    acc = lax.fori_loop(
                        0, L, red,
                        tuple(jnp.zeros((_LANE,), jnp.float32) for _ in range(EC)),
                    )
                    for c in range(EC):
                        out_v[k * _G + e, pl.ds(c * _LANE, _LANE)] = acc[c] * inv

                nk = k + _NBUF

                @pl.when(nk < NCHUNK)
                def _(b=b, nk=nk):
                    pltpu.async_copy(w_hbm.at[idx_v.at[nk]], rows_v.at[b], sems[b])

            return carry

        lax.fori_loop(0, NCHUNK // _NBUF, outer, 0)
        pltpu.sync_copy(out_v, out_hbm.at[pl.ds(wid * BPW, BPW)])

    return cbow_kernel


_K = 64


def _relayout_tc(V, E):
    grid = (V + 256 * _K - 1) // (256 * _K)

    def body(wt_ref, out_ref):
        for k in range(_K):
            x = wt_ref[:, pl.ds(256 * k, 256)]
            out_ref[pl.ds(128 * k, 128), :] = jnp.concatenate(
                [x[:, 0:128].T, x[:, 128:256].T], axis=1
            )

    rows_out = grid * 128 * _K
    return pl.pallas_call(
        body,
        grid=(grid,),
        in_specs=[pl.BlockSpec((E, 256 * _K), lambda i: (0, i))],
        out_specs=pl.BlockSpec((128 * _K, 2 * E), lambda i: (i, 0)),
        out_shape=jax.ShapeDtypeStruct((rows_out, 2 * E), jnp.float32),
    )


def kernel(data, length, W):
    B, L = data.shape
    V, E = W.shape
    d = data.astype(jnp.int32)
    d = (d & ~jnp.int32(255)) | ((d & 127) << 1) | ((d >> 7) & 1)
    data_r = d.reshape(B // _G, _G * L)
    w128 = _relayout_tc(V, E)(W.T)
    w_lin = w128.reshape(w128.shape[0] * 2, E)
    out = _cbow_sc(B, L, E, w_lin.shape[0])(data_r, w_lin)
    return out


# speedup vs baseline: 4.7665x; 1.0382x over previous
import functools

import jax
import jax.numpy as jnp
from jax import lax
from jax.experimental import pallas as pl
from jax.experimental.pallas import tpu as pltpu
from jax.experimental.pallas import tpu_sc as plsc

_LANE = 16
_NBUF = 2
_G = 2


def _cbow_sc(B, L, E, V):
    info = plsc.get_sparse_core_info()
    NC, NS = info.num_cores, info.num_subcores
    NW = NC * NS
    assert B % (NW * _G) == 0
    BPW = B // NW
    NCHUNK = BPW // _G
    EC = E // _LANE
    inv = 1.0 / L

    @functools.partial(
        pl.kernel,
        out_type=jax.ShapeDtypeStruct((B, E), jnp.float32),
        mesh=plsc.VectorSubcoreMesh(core_axis_name="c", subcore_axis_name="s"),
        compiler_params=pltpu.CompilerParams(use_tc_tiling_on_sc=False),
        scratch_types=[
            pltpu.VMEM((NCHUNK, _G * L), jnp.int32),
            pltpu.VMEM((_NBUF, _G * L, E), jnp.float32),
            pltpu.VMEM((BPW, E), jnp.float32),
        ]
        + [pltpu.SemaphoreType.DMA] * _NBUF,
    )
    def cbow_kernel(data_hbm, w_hbm, out_hbm, idx_v, rows_v, out_v, *sems):
        wid = lax.axis_index("s") * NC + lax.axis_index("c")
        base = wid * NCHUNK
        pltpu.sync_copy(data_hbm.at[pl.ds(base, NCHUNK)], idx_v)
        for b in range(_NBUF):
            pltpu.async_copy(w_hbm.at[idx_v.at[b]], rows_v.at[b], sems[b])

        def outer(g, carry):
            for b in range(_NBUF):
                k = g * _NBUF + b
                pltpu.make_async_copy(
                    w_hbm.at[idx_v.at[k]], rows_v.at[b], sems[b]
                ).wait()

                for e in range(_G):
                    def red(r, acc, b=b, e=e):
                        return tuple(
                            acc[c] + rows_v[b, e * L + r, pl.ds(c * _LANE, _LANE)]
                            for c in range(EC)
                        )

                    acc = lax.fori_loop(
                        0, L, red,
                        tuple(jnp.zeros((_LANE,), jnp.float32) for _ in range(EC)),
                    )
                    for c in range(EC):
                        out_v[k * _G + e, pl.ds(c * _LANE, _LANE)] = acc[c] * inv

                nk = k + _NBUF

                @pl.when(nk < NCHUNK)
                def _(b=b, nk=nk):
                    pltpu.async_copy(w_hbm.at[idx_v.at[nk]], rows_v.at[b], sems[b])

            return carry

        lax.fori_loop(0, NCHUNK // _NBUF, outer, 0)
        pltpu.sync_copy(out_v, out_hbm.at[pl.ds(wid * BPW, BPW)])

    return cbow_kernel


_K = 128


def _relayout_tc(V, E):
    grid = (V + 256 * _K - 1) // (256 * _K)

    def body(wt_ref, out_ref):
        for k in range(_K):
            x = wt_ref[:, pl.ds(256 * k, 256)]
            out_ref[pl.ds(128 * k, 128), :] = jnp.concatenate(
                [x[:, 0:128].T, x[:, 128:256].T], axis=1
            )

    rows_out = grid * 128 * _K
    return pl.pallas_call(
        body,
        grid=(grid,),
        in_specs=[pl.BlockSpec((E, 256 * _K), lambda i: (0, i))],
        out_specs=pl.BlockSpec((128 * _K, 2 * E), lambda i: (i, 0)),
        out_shape=jax.ShapeDtypeStruct((rows_out, 2 * E), jnp.float32),
    )


def kernel(data, length, W):
    B, L = data.shape
    V, E = W.shape
    d = data.astype(jnp.int32)
    d = (d & ~jnp.int32(255)) | ((d & 127) << 1) | ((d >> 7) & 1)
    data_r = d.reshape(B // _G, _G * L)
    w128 = _relayout_tc(V, E)(W.T)
    w_lin = w128.reshape(w128.shape[0] * 2, E)
    out = _cbow_sc(B, L, E, w_lin.shape[0])(data_r, w_lin)
    return out
